# baseline (speedup 1.0000x reference)
import dataclasses
import functools

import jax
import jax.numpy as jnp
from jax import lax
from jax.experimental import pallas as pl
from jax.experimental.pallas import tpu as pltpu
from jax.experimental.pallas import tpu_sc as plsc

B = 16384
CTX = 10
NEG = 5
D = 64
NVJ = NEG + 1
LANES = 16
NC = 2
NS = 16
NW = NC * NS
BPW = B // NW
CB = 32
NBLK = BPW // CB
NGRP = CB // LANES
NBLOCKS = B // CB
CROWS = BPW * CTX // 128
VROWS = BPW * NVJ // 128


def _sc_scores(ctx2d, vidx2d, u2, v2):
    mesh = plsc.VectorSubcoreMesh(core_axis_name="c", subcore_axis_name="s")
    cp = pltpu.CompilerParams()
    if "needs_layout_passes" in pltpu.CompilerParams.__dataclass_fields__:
        cp = dataclasses.replace(cp, needs_layout_passes=False)

    @functools.partial(
        pl.kernel,
        out_type=jax.ShapeDtypeStruct((NBLOCKS, 8, 128), jnp.float32),
        mesh=mesh,
        compiler_params=cp,
        scratch_types=[
            pltpu.VMEM((CROWS, 128), jnp.int32),
            pltpu.VMEM((VROWS, 128), jnp.int32),
            pltpu.VMEM((CB * CTX,), jnp.int32),
            pltpu.VMEM((CB * NVJ,), jnp.int32),
            pltpu.VMEM((CB * CTX, 128), jnp.float32),
            pltpu.VMEM((CB * NVJ, 128), jnp.float32),
            pltpu.VMEM((8, 128), jnp.float32),
            pltpu.SemaphoreType.DMA,
        ],
    )
    def k(u_hbm, v_hbm, cidx_hbm, vidx_hbm, out_hbm,
          cidx_v, vidx_v, cp_v, vp_v, upair_v, vpair_v, sc_v, sem):
        wid = lax.axis_index("s") * NC + lax.axis_index("c")
        pltpu.sync_copy(cidx_hbm.at[pl.ds(wid * CROWS, CROWS)], cidx_v)
        pltpu.sync_copy(vidx_hbm.at[pl.ds(wid * VROWS, VROWS)], vidx_v)
        iota = lax.broadcasted_iota(jnp.int32, (LANES,), 0)

        @pl.loop(0, NBLK)
        def _block(nb):
            for t in range(CB * CTX // LANES):
                p = nb * (CB * CTX) + t * LANES
                chunk = cidx_v[p // 128, pl.ds(p % 128, LANES)]
                cp_v[pl.ds(t * LANES, LANES)] = chunk >> 1
            for t in range(CB * NVJ // LANES):
                p = nb * (CB * NVJ) + t * LANES
                chunk = vidx_v[p // 128, pl.ds(p % 128, LANES)]
                vp_v[pl.ds(t * LANES, LANES)] = chunk >> 1
            copies = []
            for r in range(0, CB * CTX, 128):
                n = min(128, CB * CTX - r)
                copies.append(pltpu.async_copy(
                    u_hbm.at[cp_v.at[pl.ds(r, n)]],
                    upair_v.at[pl.ds(r, n)], sem))
            for r in range(0, CB * NVJ, 128):
                n = min(128, CB * NVJ - r)
                copies.append(pltpu.async_copy(
                    v_hbm.at[vp_v.at[pl.ds(r, n)]],
                    vpair_v.at[pl.ds(r, n)], sem))
            for c in copies:
                c.wait()

            @pl.loop(0, NGRP)
            def _group(g):
                scores = [None] * NVJ
                for dc in range(D // LANES):
                    usum = [None] * LANES
                    for j in range(CTX):
                        pos = nb * (CB * CTX) + g * (LANES * CTX) \
                            + iota * CTX + j
                        civ = plsc.load_gather(
                            cidx_v, [pos >> 7, pos & 127])
                        col = (civ & 1) * D + dc * LANES
                        row = g * (LANES * CTX) + iota * CTX + j
                        for dd in range(LANES):
                            val = plsc.load_gather(upair_v, [row, col + dd])
                            usum[dd] = val if j == 0 else usum[dd] + val
                    for dd in range(LANES):
                        usum[dd] = usum[dd] * (1.0 / CTX)
                    for j6 in range(NVJ):
                        pos = nb * (CB * NVJ) + g * (LANES * NVJ) \
                            + iota * NVJ + j6
                        viv = plsc.load_gather(
                            vidx_v, [pos >> 7, pos & 127])
                        vcol = (viv & 1) * D + dc * LANES
                        vrow = g * (LANES * NVJ) + iota * NVJ + j6
                        acc = scores[j6]
                        for dd in range(LANES):
                            vval = plsc.load_gather(vpair_v,
                                                    [vrow, vcol + dd])
                            t = usum[dd] * vval
                            acc = t if (dc == 0 and dd == 0) else acc + t
                        scores[j6] = acc
                for j6 in range(NVJ):
                    s = scores[j6] if j6 == 0 else -scores[j6]
                    sc_v[j6, pl.ds(g * LANES, LANES)] = s

            pltpu.sync_copy(sc_v, out_hbm.at[wid * NBLK + nb])

    return k(u2, v2, ctx2d, vidx2d)


def _tc_loss(scores):
    def body(p_ref, o_ref):
        o_ref[...] = -jnp.sum(jax.nn.log_sigmoid(p_ref[...]))[None, None]

    out = pl.pallas_call(
        body,
        in_specs=[pl.BlockSpec(scores.shape, lambda: (0, 0))],
        out_specs=pl.BlockSpec((1, 1), lambda: (0, 0)),
        out_shape=jax.ShapeDtypeStruct((1, 1), jnp.float32),
    )(scores)
    return out[0, 0]


def kernel(context, target, negatives, u_table, v_table):
    u2 = u_table[:1000000].reshape(500000, 128)
    v2 = v_table[:1000000].reshape(500000, 128)
    ctx2d = context.astype(jnp.int32).reshape(B * CTX // 128, 128)
    vidx2d = jnp.concatenate(
        [target[:, None], negatives], axis=1).astype(jnp.int32).reshape(
            B * NVJ // 128, 128)
    raw = _sc_scores(ctx2d, vidx2d, u2, v2)
    scores = raw[:, :NVJ, :CB].reshape(B * NVJ // 128, 128)
    return _tc_loss(scores)
